# Initial kernel scaffold; baseline (speedup 1.0000x reference)
import jax
import jax.numpy as jnp
from jax.experimental import pallas as pl


def kernel(x, proj_w, proj_b, se_w1, se_b1, se_w2, se_b2, sp_w, sp_b):
    raise NotImplementedError("write your pallas kernel here")


# speedup vs baseline: 7.3752x; 7.3752x over previous
import functools

import jax
import jax.numpy as jnp
from jax.experimental import pallas as pl
from jax.experimental.pallas import tpu as pltpu


def _fused_kernel(H, W, x_ref, wT_ref, pb_ref, w1T_ref, b1_ref, w2T_ref,
                  b2_ref, wk_ref, bk_ref, o_ref, spad):
    HW = H * W
    PAD = 3 * W + 3

    xb = x_ref[0]

    yT = (jnp.dot(wT_ref[...], xb, preferred_element_type=jnp.float32)
          + pb_ref[...])

    meanC = jnp.sum(yT, axis=1, keepdims=True) * (1.0 / float(HW))

    z1 = jnp.maximum(
        jnp.dot(w1T_ref[...], meanC, preferred_element_type=jnp.float32)
        + b1_ref[...], 0.0)
    attC = jax.nn.sigmoid(
        jnp.dot(w2T_ref[...], z1, preferred_element_type=jnp.float32)
        + b2_ref[...])

    xcw = yT * attC

    R = jnp.dot(wk_ref[...], xcw, preferred_element_type=jnp.float32)

    spad[:, 0:PAD] = jnp.zeros((49, PAD), jnp.float32)
    spad[:, PAD + HW:] = jnp.zeros((49, PAD), jnp.float32)
    spad[:, PAD:PAD + HW] = R

    wl = jax.lax.broadcasted_iota(jnp.int32, (1, HW), 1)
    wl = (wl & (W - 1)) if (W & (W - 1)) == 0 else (wl % W)

    acc = jnp.zeros((1, HW), jnp.float32) + bk_ref[...]
    for kj in range(7):
        inner = None
        for ki in range(7):
            t = ki * 7 + kj
            off = PAD + (ki - 3) * W + (kj - 3)
            sl = spad[t:t + 1, off:off + HW]
            inner = sl if inner is None else inner + sl
        if kj < 3:
            inner = jnp.where(wl >= (3 - kj), inner, 0.0)
        elif kj > 3:
            inner = jnp.where(wl < (W + 3 - kj), inner, 0.0)
        acc = acc + inner

    att_s = jax.nn.sigmoid(acc)
    o_ref[0] = xcw * att_s


def kernel(x, proj_w, proj_b, se_w1, se_b1, se_w2, se_b2, sp_w, sp_b):
    B, Cin, H, W = x.shape
    Cout = proj_w.shape[1]
    Cr = se_w1.shape[1]
    HW = H * W
    PAD = 3 * W + 3

    x3 = x.reshape(B, Cin, HW)

    out = pl.pallas_call(
        functools.partial(_fused_kernel, H, W),
        out_shape=jax.ShapeDtypeStruct((B, Cout, HW), jnp.float32),
        grid=(B,),
        in_specs=[
            pl.BlockSpec((1, Cin, HW), lambda b: (b, 0, 0)),
            pl.BlockSpec((Cout, Cin), lambda b: (0, 0)),
            pl.BlockSpec((Cout, 1), lambda b: (0, 0)),
            pl.BlockSpec((Cr, Cout), lambda b: (0, 0)),
            pl.BlockSpec((Cr, 1), lambda b: (0, 0)),
            pl.BlockSpec((Cout, Cr), lambda b: (0, 0)),
            pl.BlockSpec((Cout, 1), lambda b: (0, 0)),
            pl.BlockSpec((49, Cout), lambda b: (0, 0)),
            pl.BlockSpec((1, 1), lambda b: (0, 0)),
        ],
        out_specs=pl.BlockSpec((1, Cout, HW), lambda b: (b, 0, 0)),
        scratch_shapes=[pltpu.VMEM((49, 2 * PAD + HW), jnp.float32)],
        compiler_params=pltpu.CompilerParams(
            dimension_semantics=("parallel",),
            vmem_limit_bytes=64 * 1024 * 1024),
        cost_estimate=pl.CostEstimate(
            flops=2 * B * HW * Cout * (Cin + 49),
            transcendentals=B * (HW + Cout),
            bytes_accessed=4 * (B * HW * (Cin + Cout)
                                + Cout * (Cin + 2 * Cr + 49)),
        ),
    )(x3,
      proj_w.T,
      proj_b.reshape(Cout, 1),
      se_w1.T,
      se_b1.reshape(Cr, 1),
      se_w2.T,
      se_b2.reshape(Cout, 1),
      sp_w.reshape(49, Cout),
      sp_b.reshape(1, 1))

    return out.reshape(B, Cout, H, W)
